# baseline (speedup 1.0000x reference)
import functools

import jax
import jax.numpy as jnp
from jax import lax
from jax.experimental import pallas as pl
from jax.experimental.pallas import tpu as pltpu
from jax.experimental.pallas import tpu_sc as plsc

B = 128
T = 4096
F = 16
SEASON_LEN = 24
C_OUT = 19
LANES = 16
NUM_CORES = 2
NUM_SUBCORES = 16
NUM_WORKERS = NUM_CORES * NUM_SUBCORES
BATCH_PER_WORKER = B // NUM_WORKERS
NUM_CHUNKS = T // LANES
TGRP = T // 128


def _pow_e(base, e):
    r = jnp.ones((LANES,), jnp.float32)
    b = base
    for bit in range(5):
        m = ((e >> bit) & 1) == 1
        r = jnp.where(m, r * b, r)
        b = b * b
    return r


def _hw_body(in_hbm, a_hbm, g_hbm, out_hbm, in_v, y5_v, l5_v, s5_v,
             a_v, g_v):
    cid = lax.axis_index("c")
    sid = lax.axis_index("s")
    wid = sid * NUM_CORES + cid

    pltpu.sync_copy(a_hbm, a_v)
    pltpu.sync_copy(g_hbm, g_v)
    av = a_v[...]
    gv = g_v[...]
    oma = 1.0 - av
    omg = 1.0 - gv
    iota = lax.iota(jnp.int32, LANES)
    pw = _pow_e(oma, iota + 1)
    ipw = _pow_e(1.0 / oma, iota + 1)
    e_mix = jnp.maximum(0, iota - (SEASON_LEN - LANES - 1))
    pw_mix = _pow_e(oma, e_mix)
    ipw_mix = _pow_e(1.0 / oma, e_mix)
    act = iota >= (SEASON_LEN - LANES)
    m8 = iota < (SEASON_LEN - LANES)
    rot8_keys = (iota + 8) & 15
    half = iota < 8

    for bi_ in range(BATCH_PER_WORKER):
        b = wid * BATCH_PER_WORKER + bi_
        bg = b // 8
        bi = b % 8
        pltpu.sync_copy(in_hbm.at[b], in_v)
        for c in range(F):
            pltpu.sync_copy(
                in_v.at[c // 8, :, c % 8:c % 8 + 1, :],
                out_hbm.at[c + 1, bg, :, pl.ds(bi, 1), :])

        p0 = in_v[0, 0, 0, pl.ds(0, LANES)]
        p1 = in_v[0, 0, 0, pl.ds(LANES, LANES)]
        init = (jnp.sum(p0) + jnp.sum(jnp.where(m8, p1, 0.0))) * (
            1.0 / SEASON_LEN)
        init_v = lax.broadcast(init, (LANES,))
        zeros = jnp.zeros((LANES,), jnp.float32)
        y5_v[0, 0, pl.ds(0, LANES)] = p0
        l5_v[0, 0, pl.ds(0, LANES)] = init_v
        s5_v[0, 0, pl.ds(0, LANES)] = zeros

        z1 = jnp.where(act, p1, 0.0)
        cs1 = plsc.cumsum(av * z1 * ipw_mix)
        l1 = pw_mix * (init + cs1)
        s1 = jnp.where(act, gv * (p1 - l1), 0.0)
        y1 = p1 - s1
        y5_v[0, 0, pl.ds(LANES, LANES)] = y1
        l5_v[0, 0, pl.ds(LANES, LANES)] = l1
        s5_v[0, 0, pl.ds(LANES, LANES)] = s1

        def chunk(i, carry):
            lprev, s_p1, s_p2 = carry
            t0 = LANES * i
            tg = t0 >> 7
            ti = t0 & 127
            p = in_v[0, tg, 0, pl.ds(ti, LANES)]
            u = jnp.where(half, s_p1, s_p2)
            _, slag = plsc.sort_key_val(rot8_keys, u)
            w = av * (p - slag) * ipw
            cs = plsc.cumsum(w)
            l = pw * (lprev + cs)
            s = gv * (p - l) + omg * slag
            y = p - s
            y5_v[tg, 0, pl.ds(ti, LANES)] = y
            l5_v[tg, 0, pl.ds(ti, LANES)] = l
            s5_v[tg, 0, pl.ds(ti, LANES)] = s
            return (jnp.sum(jnp.where(iota == LANES - 1, l, 0.0)), s, s_p1)

        lax.fori_loop(
            2, NUM_CHUNKS, chunk,
            (jnp.sum(jnp.where(iota == LANES - 1, l1, 0.0)), s1, zeros))

        pltpu.sync_copy(y5_v, out_hbm.at[0, bg, :, pl.ds(bi, 1), :])
        pltpu.sync_copy(l5_v, out_hbm.at[C_OUT - 2, bg, :, pl.ds(bi, 1), :])
        pltpu.sync_copy(s5_v, out_hbm.at[C_OUT - 1, bg, :, pl.ds(bi, 1), :])


def kernel(inputs, alpha, gamma):
    mesh = plsc.VectorSubcoreMesh(
        core_axis_name="c", subcore_axis_name="s",
        num_cores=NUM_CORES, num_subcores=NUM_SUBCORES)
    hw = functools.partial(
        pl.kernel,
        out_type=jax.ShapeDtypeStruct((C_OUT, B // 8, TGRP, 8, 128),
                                      jnp.float32),
        mesh=mesh,
        scratch_types=[
            pltpu.VMEM((F // 8, TGRP, 8, 128), jnp.float32),
            pltpu.VMEM((TGRP, 1, 128), jnp.float32),
            pltpu.VMEM((TGRP, 1, 128), jnp.float32),
            pltpu.VMEM((TGRP, 1, 128), jnp.float32),
            pltpu.VMEM((LANES,), jnp.float32),
            pltpu.VMEM((LANES,), jnp.float32),
        ],
        compiler_params=pltpu.CompilerParams(
            needs_layout_passes=False, use_tc_tiling_on_sc=False),
    )(_hw_body)
    a16 = jnp.broadcast_to(alpha.astype(jnp.float32), (LANES,))
    g16 = jnp.broadcast_to(gamma.astype(jnp.float32), (LANES,))
    in5 = jnp.transpose(
        jnp.reshape(jnp.transpose(inputs, (0, 2, 1)),
                    (B, F // 8, 8, TGRP, 128)),
        (0, 1, 3, 2, 4))
    out5 = hw(in5, a16, g16)
    out = jnp.reshape(jnp.transpose(out5, (1, 3, 2, 4, 0)),
                      (B, T, C_OUT))
    return out


# speedup vs baseline: 1.0523x; 1.0523x over previous
import functools

import jax
import jax.numpy as jnp
from jax import lax
from jax.experimental import pallas as pl
from jax.experimental.pallas import tpu as pltpu
from jax.experimental.pallas import tpu_sc as plsc

B = 128
T = 4096
F = 16
SEASON_LEN = 24
C_OUT = 19
LANES = 16
NUM_CORES = 2
NUM_SUBCORES = 16
NUM_WORKERS = NUM_CORES * NUM_SUBCORES
BATCH_PER_WORKER = B // NUM_WORKERS
NUM_CHUNKS = T // LANES
TGRP = T // 128


def _pow_e(base, e):
    r = jnp.ones((LANES,), jnp.float32)
    b = base
    for bit in range(5):
        m = ((e >> bit) & 1) == 1
        r = jnp.where(m, r * b, r)
        b = b * b
    return r


def _hw_body(in_hbm, a_hbm, g_hbm, out_hbm, in_v, y5_v, l5_v, s5_v,
             a_v, g_v, csem, osem0, osem1):
    cid = lax.axis_index("c")
    sid = lax.axis_index("s")
    wid = sid * NUM_CORES + cid

    pltpu.sync_copy(a_hbm, a_v)
    pltpu.sync_copy(g_hbm, g_v)
    av = a_v[...]
    gv = g_v[...]
    oma = 1.0 - av
    omg = 1.0 - gv
    iota = lax.iota(jnp.int32, LANES)
    pw = _pow_e(oma, iota + 1)
    ipw = _pow_e(1.0 / oma, iota + 1)
    e_mix = jnp.maximum(0, iota - (SEASON_LEN - LANES - 1))
    pw_mix = _pow_e(oma, e_mix)
    ipw_mix = _pow_e(1.0 / oma, e_mix)
    act = iota >= (SEASON_LEN - LANES)
    m8 = iota < (SEASON_LEN - LANES)
    rot8_keys = (iota + 8) & 15
    half = iota < 8

    ch_descs = []
    out_descs = [[], []]
    osems = [osem0, osem1]
    for bi_ in range(BATCH_PER_WORKER):
        b = wid * BATCH_PER_WORKER + bi_
        bg = b // 8
        bi = b % 8
        par = bi_ & 1
        for d in ch_descs:
            d.wait()
        ch_descs = []
        pltpu.sync_copy(in_hbm.at[b], in_v)
        for c in range(F):
            ch_descs.append(pltpu.async_copy(
                in_v.at[c // 8, :, c % 8:c % 8 + 1, :],
                out_hbm.at[c + 1, bg, :, pl.ds(bi, 1), :], csem))
        for d in out_descs[par]:
            d.wait()
        out_descs[par] = []

        p0 = in_v[0, 0, 0, pl.ds(0, LANES)]
        p1 = in_v[0, 0, 0, pl.ds(LANES, LANES)]
        init = (jnp.sum(p0) + jnp.sum(jnp.where(m8, p1, 0.0))) * (
            1.0 / SEASON_LEN)
        init_v = lax.broadcast(init, (LANES,))
        zeros = jnp.zeros((LANES,), jnp.float32)
        y5_v[par, 0, 0, pl.ds(0, LANES)] = p0
        l5_v[par, 0, 0, pl.ds(0, LANES)] = init_v
        s5_v[par, 0, 0, pl.ds(0, LANES)] = zeros

        z1 = jnp.where(act, p1, 0.0)
        cs1 = plsc.cumsum(av * z1 * ipw_mix)
        l1 = pw_mix * (init + cs1)
        s1 = jnp.where(act, gv * (p1 - l1), 0.0)
        y1 = p1 - s1
        y5_v[par, 0, 0, pl.ds(LANES, LANES)] = y1
        l5_v[par, 0, 0, pl.ds(LANES, LANES)] = l1
        s5_v[par, 0, 0, pl.ds(LANES, LANES)] = s1

        def chunk(i, carry):
            lprev, s_p1, s_p2 = carry
            t0 = LANES * i
            tg = t0 >> 7
            ti = t0 & 127
            p = in_v[0, tg, 0, pl.ds(ti, LANES)]
            u = jnp.where(half, s_p1, s_p2)
            _, slag = plsc.sort_key_val(rot8_keys, u)
            w = av * (p - slag) * ipw
            cs = plsc.cumsum(w)
            l = pw * (lprev + cs)
            s = gv * (p - l) + omg * slag
            y = p - s
            y5_v[par, tg, 0, pl.ds(ti, LANES)] = y
            l5_v[par, tg, 0, pl.ds(ti, LANES)] = l
            s5_v[par, tg, 0, pl.ds(ti, LANES)] = s
            return (jnp.sum(jnp.where(iota == LANES - 1, l, 0.0)), s, s_p1)

        lax.fori_loop(
            2, NUM_CHUNKS, chunk,
            (jnp.sum(jnp.where(iota == LANES - 1, l1, 0.0)), s1, zeros))

        out_descs[par] = [
            pltpu.async_copy(
                y5_v.at[par], out_hbm.at[0, bg, :, pl.ds(bi, 1), :],
                osems[par]),
            pltpu.async_copy(
                l5_v.at[par],
                out_hbm.at[C_OUT - 2, bg, :, pl.ds(bi, 1), :], osems[par]),
            pltpu.async_copy(
                s5_v.at[par],
                out_hbm.at[C_OUT - 1, bg, :, pl.ds(bi, 1), :], osems[par]),
        ]

    for d in ch_descs:
        d.wait()
    for par in (0, 1):
        for d in out_descs[par]:
            d.wait()


def kernel(inputs, alpha, gamma):
    mesh = plsc.VectorSubcoreMesh(
        core_axis_name="c", subcore_axis_name="s",
        num_cores=NUM_CORES, num_subcores=NUM_SUBCORES)
    hw = functools.partial(
        pl.kernel,
        out_type=jax.ShapeDtypeStruct((C_OUT, B // 8, TGRP, 8, 128),
                                      jnp.float32),
        mesh=mesh,
        scratch_types=[
            pltpu.VMEM((F // 8, TGRP, 8, 128), jnp.float32),
            pltpu.VMEM((2, TGRP, 1, 128), jnp.float32),
            pltpu.VMEM((2, TGRP, 1, 128), jnp.float32),
            pltpu.VMEM((2, TGRP, 1, 128), jnp.float32),
            pltpu.VMEM((LANES,), jnp.float32),
            pltpu.VMEM((LANES,), jnp.float32),
            pltpu.SemaphoreType.DMA,
            pltpu.SemaphoreType.DMA,
            pltpu.SemaphoreType.DMA,
        ],
        compiler_params=pltpu.CompilerParams(
            needs_layout_passes=False, use_tc_tiling_on_sc=False),
    )(_hw_body)
    a16 = jnp.broadcast_to(alpha.astype(jnp.float32), (LANES,))
    g16 = jnp.broadcast_to(gamma.astype(jnp.float32), (LANES,))
    in5 = jnp.transpose(
        jnp.reshape(jnp.transpose(inputs, (0, 2, 1)),
                    (B, F // 8, 8, TGRP, 128)),
        (0, 1, 3, 2, 4))
    out5 = hw(in5, a16, g16)
    out = jnp.reshape(jnp.transpose(out5, (1, 3, 2, 4, 0)),
                      (B, T, C_OUT))
    return out


# speedup vs baseline: 1.3282x; 1.2623x over previous
import functools

import jax
import jax.numpy as jnp
from jax import lax
from jax.experimental import pallas as pl
from jax.experimental.pallas import tpu as pltpu
from jax.experimental.pallas import tpu_sc as plsc

B = 128
T = 4096
F = 16
SEASON_LEN = 24
C_OUT = 19
LANES = 16
NUM_CORES = 2
NUM_SUBCORES = 16
NUM_WORKERS = NUM_CORES * NUM_SUBCORES
BATCH_PER_WORKER = B // NUM_WORKERS
NUM_CHUNKS = T // LANES
TGRP = T // 128
QCH = 4
NWAVES = BATCH_PER_WORKER * (F // QCH)


def _pow_e(base, e):
    r = jnp.ones((LANES,), jnp.float32)
    b = base
    for bit in range(5):
        m = ((e >> bit) & 1) == 1
        r = jnp.where(m, r * b, r)
        b = b * b
    return r


def _hw_body(in_hbm, a_hbm, g_hbm, out_hbm, p5_v, y5_v, l5_v, s5_v, ch_v,
             a_v, g_v, isem0, isem1, csem0, csem1, osem):
    cid = lax.axis_index("c")
    sid = lax.axis_index("s")
    wid = sid * NUM_CORES + cid

    pltpu.sync_copy(a_hbm, a_v)
    pltpu.sync_copy(g_hbm, g_v)
    av = a_v[...]
    gv = g_v[...]
    oma = 1.0 - av
    omg = 1.0 - gv
    iota = lax.iota(jnp.int32, LANES)
    pw = _pow_e(oma, iota + 1)
    ipw = _pow_e(1.0 / oma, iota + 1)
    e_mix = jnp.maximum(0, iota - (SEASON_LEN - LANES - 1))
    pw_mix = _pow_e(oma, e_mix)
    ipw_mix = _pow_e(1.0 / oma, e_mix)
    act = iota >= (SEASON_LEN - LANES)
    m8 = iota < (SEASON_LEN - LANES)
    rot8_keys = (iota + 8) & 15
    half = iota < 8
    zeros = jnp.zeros((LANES,), jnp.float32)

    batches = [wid * BATCH_PER_WORKER + j for j in range(BATCH_PER_WORKER)]

    for j, b in enumerate(batches):
        pltpu.sync_copy(in_hbm.at[b, 0, :, 0:1, :], p5_v.at[j])

    carries = []
    for j in range(BATCH_PER_WORKER):
        p0 = p5_v[j, 0, 0, pl.ds(0, LANES)]
        p1 = p5_v[j, 0, 0, pl.ds(LANES, LANES)]
        init = (jnp.sum(p0) + jnp.sum(jnp.where(m8, p1, 0.0))) * (
            1.0 / SEASON_LEN)
        init_v = lax.broadcast(init, (LANES,))
        y5_v[j, 0, 0, pl.ds(0, LANES)] = p0
        l5_v[j, 0, 0, pl.ds(0, LANES)] = init_v
        s5_v[j, 0, 0, pl.ds(0, LANES)] = zeros
        z1 = jnp.where(act, p1, 0.0)
        cs1 = plsc.cumsum(av * z1 * ipw_mix)
        l1 = pw_mix * (init + cs1)
        s1 = jnp.where(act, gv * (p1 - l1), 0.0)
        y1 = p1 - s1
        y5_v[j, 0, 0, pl.ds(LANES, LANES)] = y1
        l5_v[j, 0, 0, pl.ds(LANES, LANES)] = l1
        s5_v[j, 0, 0, pl.ds(LANES, LANES)] = s1
        carries += [jnp.sum(jnp.where(iota == LANES - 1, l1, 0.0)), s1,
                    zeros]

    def chunk(i, carry):
        t0 = LANES * i
        tg = t0 >> 7
        ti = t0 & 127
        out = []
        for j in range(BATCH_PER_WORKER):
            lprev, s_p1, s_p2 = carry[3 * j:3 * j + 3]
            p = p5_v[j, tg, 0, pl.ds(ti, LANES)]
            u = jnp.where(half, s_p1, s_p2)
            _, slag = plsc.sort_key_val(rot8_keys, u)
            w = av * (p - slag) * ipw
            cs = plsc.cumsum(w)
            l = pw * (lprev + cs)
            s = gv * (p - l) + omg * slag
            y = p - s
            y5_v[j, tg, 0, pl.ds(ti, LANES)] = y
            l5_v[j, tg, 0, pl.ds(ti, LANES)] = l
            s5_v[j, tg, 0, pl.ds(ti, LANES)] = s
            out += [jnp.sum(jnp.where(iota == LANES - 1, l, 0.0)), s, s_p1]
        return tuple(out)

    lax.fori_loop(2, NUM_CHUNKS, chunk, tuple(carries))

    out_descs = []
    for j, b in enumerate(batches):
        bg = b // 8
        bi = b % 8
        out_descs += [
            pltpu.async_copy(
                y5_v.at[j], out_hbm.at[0, bg, :, pl.ds(bi, 1), :], osem),
            pltpu.async_copy(
                l5_v.at[j], out_hbm.at[C_OUT - 2, bg, :, pl.ds(bi, 1), :],
                osem),
            pltpu.async_copy(
                s5_v.at[j], out_hbm.at[C_OUT - 1, bg, :, pl.ds(bi, 1), :],
                osem),
        ]

    isems = [isem0, isem1]
    csems = [csem0, csem1]

    def fire_in(w):
        b = batches[w // (F // QCH)]
        c0 = (w % (F // QCH)) * QCH
        return pltpu.async_copy(
            in_hbm.at[b, c0 // 8, :, pl.ds(c0 % 8, QCH), :],
            ch_v.at[w & 1], isems[w & 1])

    in_descs = {0: fire_in(0)}
    ch_descs = {0: [], 1: []}
    for w in range(NWAVES):
        buf = w & 1
        if w + 1 < NWAVES:
            for d in ch_descs[1 - buf]:
                d.wait()
            ch_descs[1 - buf] = []
            in_descs[1 - buf] = fire_in(w + 1)
        in_descs[buf].wait()
        b = batches[w // (F // QCH)]
        bg = b // 8
        bi = b % 8
        c0 = (w % (F // QCH)) * QCH
        ch_descs[buf] = [
            pltpu.async_copy(
                ch_v.at[buf, :, pl.ds(q, 1), :],
                out_hbm.at[c0 + q + 1, bg, :, pl.ds(bi, 1), :], csems[buf])
            for q in range(QCH)
        ]

    for buf in (0, 1):
        for d in ch_descs[buf]:
            d.wait()
    for d in out_descs:
        d.wait()


def kernel(inputs, alpha, gamma):
    mesh = plsc.VectorSubcoreMesh(
        core_axis_name="c", subcore_axis_name="s",
        num_cores=NUM_CORES, num_subcores=NUM_SUBCORES)
    hw = functools.partial(
        pl.kernel,
        out_type=jax.ShapeDtypeStruct((C_OUT, B // 8, TGRP, 8, 128),
                                      jnp.float32),
        mesh=mesh,
        scratch_types=[
            pltpu.VMEM((BATCH_PER_WORKER, TGRP, 1, 128), jnp.float32),
            pltpu.VMEM((BATCH_PER_WORKER, TGRP, 1, 128), jnp.float32),
            pltpu.VMEM((BATCH_PER_WORKER, TGRP, 1, 128), jnp.float32),
            pltpu.VMEM((BATCH_PER_WORKER, TGRP, 1, 128), jnp.float32),
            pltpu.VMEM((2, TGRP, QCH, 128), jnp.float32),
            pltpu.VMEM((LANES,), jnp.float32),
            pltpu.VMEM((LANES,), jnp.float32),
            pltpu.SemaphoreType.DMA,
            pltpu.SemaphoreType.DMA,
            pltpu.SemaphoreType.DMA,
            pltpu.SemaphoreType.DMA,
            pltpu.SemaphoreType.DMA,
        ],
        compiler_params=pltpu.CompilerParams(
            needs_layout_passes=False, use_tc_tiling_on_sc=False),
    )(_hw_body)
    a16 = jnp.broadcast_to(alpha.astype(jnp.float32), (LANES,))
    g16 = jnp.broadcast_to(gamma.astype(jnp.float32), (LANES,))
    in5 = jnp.transpose(
        jnp.reshape(jnp.transpose(inputs, (0, 2, 1)),
                    (B, F // 8, 8, TGRP, 128)),
        (0, 1, 3, 2, 4))
    out5 = hw(in5, a16, g16)
    out = jnp.reshape(jnp.transpose(out5, (1, 3, 2, 4, 0)),
                      (B, T, C_OUT))
    return out


# speedup vs baseline: 1.5627x; 1.1765x over previous
import functools

import jax
import jax.numpy as jnp
from jax import lax
from jax.experimental import pallas as pl
from jax.experimental.pallas import tpu as pltpu
from jax.experimental.pallas import tpu_sc as plsc

B = 128
T = 4096
F = 16
SEASON_LEN = 24
C_OUT = 19
LANES = 16
NUM_CORES = 2
NUM_SUBCORES = 16
NUM_WORKERS = NUM_CORES * NUM_SUBCORES
BATCH_PER_WORKER = B // NUM_WORKERS
NUM_CHUNKS = T // LANES
TGRP = T // 128
QCH = 4
NWAVES = BATCH_PER_WORKER * (F // QCH)


def _pow_e(base, e):
    r = jnp.ones((LANES,), jnp.float32)
    b = base
    for bit in range(5):
        m = ((e >> bit) & 1) == 1
        r = jnp.where(m, r * b, r)
        b = b * b
    return r


def _hw_body(in_hbm, a_hbm, g_hbm, out_hbm, p5_v, y5_v, l5_v, s5_v, ch_v,
             a_v, g_v, isem0, isem1, isem2, csem0, csem1, csem2, osem):
    cid = lax.axis_index("c")
    sid = lax.axis_index("s")
    wid = sid * NUM_CORES + cid

    pltpu.sync_copy(a_hbm, a_v)
    pltpu.sync_copy(g_hbm, g_v)
    av = a_v[...]
    gv = g_v[...]
    oma = 1.0 - av
    omg = 1.0 - gv
    iota = lax.iota(jnp.int32, LANES)
    pw = _pow_e(oma, iota + 1)
    ipw = _pow_e(1.0 / oma, iota + 1)
    avipw = av * ipw
    e_mix = jnp.maximum(0, iota - (SEASON_LEN - LANES - 1))
    pw_mix = _pow_e(oma, e_mix)
    ipw_mix = _pow_e(1.0 / oma, e_mix)
    act = iota >= (SEASON_LEN - LANES)
    m8 = iota < (SEASON_LEN - LANES)
    rot8_keys = (iota + 8) & 15
    half = iota < 8
    zeros = jnp.zeros((LANES,), jnp.float32)

    batches = [wid * BATCH_PER_WORKER + j for j in range(BATCH_PER_WORKER)]

    for j, b in enumerate(batches):
        pltpu.sync_copy(in_hbm.at[b, 0, :, 0:1, :], p5_v.at[j])

    carries = []
    for j in range(BATCH_PER_WORKER):
        p0 = p5_v[j, 0, 0, pl.ds(0, LANES)]
        p1 = p5_v[j, 0, 0, pl.ds(LANES, LANES)]
        init = (jnp.sum(p0) + jnp.sum(jnp.where(m8, p1, 0.0))) * (
            1.0 / SEASON_LEN)
        init_v = lax.broadcast(init, (LANES,))
        y5_v[j, 0, 0, pl.ds(0, LANES)] = p0
        l5_v[j, 0, 0, pl.ds(0, LANES)] = init_v
        s5_v[j, 0, 0, pl.ds(0, LANES)] = zeros
        z1 = jnp.where(act, p1, 0.0)
        cs1 = plsc.cumsum(av * z1 * ipw_mix)
        l1 = pw_mix * (init + cs1)
        s1 = jnp.where(act, gv * (p1 - l1), 0.0)
        y1 = p1 - s1
        y5_v[j, 0, 0, pl.ds(LANES, LANES)] = y1
        l5_v[j, 0, 0, pl.ds(LANES, LANES)] = l1
        s5_v[j, 0, 0, pl.ds(LANES, LANES)] = s1
        carries += [l1[LANES - 1], s1, zeros]

    isems = [isem0, isem1, isem2]
    csems = [csem0, csem1, csem2]

    def fire_in(w):
        b_ = batches[w // (F // QCH)]
        c0 = (w % (F // QCH)) * QCH
        return pltpu.async_copy(
            in_hbm.at[b_, c0 // 8, :, pl.ds(c0 % 8, QCH), :],
            ch_v.at[w % 3], isems[w % 3])

    in_descs = {0: fire_in(0), 1: fire_in(1)}

    def chunk(i, carry):
        t0 = LANES * i
        tg = t0 >> 7
        ti = t0 & 127
        out = []
        for j in range(BATCH_PER_WORKER):
            lprev, s_p1, s_p2 = carry[3 * j:3 * j + 3]
            p = p5_v[j, tg, 0, pl.ds(ti, LANES)]
            u = jnp.where(half, s_p1, s_p2)
            _, slag = plsc.sort_key_val(rot8_keys, u)
            w = (p - slag) * avipw
            cs = plsc.cumsum(w)
            l = pw * (lprev + cs)
            s = gv * (p - l) + omg * slag
            y = p - s
            y5_v[j, tg, 0, pl.ds(ti, LANES)] = y
            l5_v[j, tg, 0, pl.ds(ti, LANES)] = l
            s5_v[j, tg, 0, pl.ds(ti, LANES)] = s
            out += [l[LANES - 1], s, s_p1]
        return tuple(out)

    lax.fori_loop(2, NUM_CHUNKS, chunk, tuple(carries))

    out_descs = []
    for j, b in enumerate(batches):
        bg = b // 8
        bi = b % 8
        out_descs += [
            pltpu.async_copy(
                y5_v.at[j], out_hbm.at[0, bg, :, pl.ds(bi, 1), :], osem),
            pltpu.async_copy(
                l5_v.at[j], out_hbm.at[C_OUT - 2, bg, :, pl.ds(bi, 1), :],
                osem),
            pltpu.async_copy(
                s5_v.at[j], out_hbm.at[C_OUT - 1, bg, :, pl.ds(bi, 1), :],
                osem),
        ]

    ch_descs = {0: [], 1: [], 2: []}
    for w in range(NWAVES):
        buf = w % 3
        if w + 2 < NWAVES:
            nb = (w + 2) % 3
            for d in ch_descs[nb]:
                d.wait()
            ch_descs[nb] = []
            in_descs[nb] = fire_in(w + 2)
        in_descs[buf].wait()
        b = batches[w // (F // QCH)]
        bg = b // 8
        bi = b % 8
        c0 = (w % (F // QCH)) * QCH
        ch_descs[buf] = [
            pltpu.async_copy(
                ch_v.at[buf, :, pl.ds(q, 1), :],
                out_hbm.at[c0 + q + 1, bg, :, pl.ds(bi, 1), :], csems[buf])
            for q in range(QCH)
        ]

    for buf in (0, 1, 2):
        for d in ch_descs[buf]:
            d.wait()
    for d in out_descs:
        d.wait()


def kernel(inputs, alpha, gamma):
    mesh = plsc.VectorSubcoreMesh(
        core_axis_name="c", subcore_axis_name="s",
        num_cores=NUM_CORES, num_subcores=NUM_SUBCORES)
    hw = functools.partial(
        pl.kernel,
        out_type=jax.ShapeDtypeStruct((C_OUT, B // 8, TGRP, 8, 128),
                                      jnp.float32),
        mesh=mesh,
        scratch_types=[
            pltpu.VMEM((BATCH_PER_WORKER, TGRP, 1, 128), jnp.float32),
            pltpu.VMEM((BATCH_PER_WORKER, TGRP, 1, 128), jnp.float32),
            pltpu.VMEM((BATCH_PER_WORKER, TGRP, 1, 128), jnp.float32),
            pltpu.VMEM((BATCH_PER_WORKER, TGRP, 1, 128), jnp.float32),
            pltpu.VMEM((3, TGRP, QCH, 128), jnp.float32),
            pltpu.VMEM((LANES,), jnp.float32),
            pltpu.VMEM((LANES,), jnp.float32),
            pltpu.SemaphoreType.DMA,
            pltpu.SemaphoreType.DMA,
            pltpu.SemaphoreType.DMA,
            pltpu.SemaphoreType.DMA,
            pltpu.SemaphoreType.DMA,
            pltpu.SemaphoreType.DMA,
            pltpu.SemaphoreType.DMA,
        ],
        compiler_params=pltpu.CompilerParams(
            needs_layout_passes=False, use_tc_tiling_on_sc=False),
    )(_hw_body)
    a16 = jnp.broadcast_to(alpha.astype(jnp.float32), (LANES,))
    g16 = jnp.broadcast_to(gamma.astype(jnp.float32), (LANES,))
    in5 = jnp.transpose(
        jnp.reshape(jnp.transpose(inputs, (0, 2, 1)),
                    (B, F // 8, 8, TGRP, 128)),
        (0, 1, 3, 2, 4))
    out5 = hw(in5, a16, g16)
    out = jnp.reshape(jnp.transpose(out5, (1, 3, 2, 4, 0)),
                      (B, T, C_OUT))
    return out
